# Initial kernel scaffold; baseline (speedup 1.0000x reference)
import jax
import jax.numpy as jnp
from jax.experimental import pallas as pl


def kernel(word_ids, seg_ids, posi_ids, age_ids, gender_ids, word_table, seg_table, age_table, gender_table, posi_table, gamma, beta):
    raise NotImplementedError("write your pallas kernel here")


# speedup vs baseline: 6.5616x; 6.5616x over previous
import functools

import jax
import jax.numpy as jnp
from jax import lax
from jax.experimental import pallas as pl
from jax.experimental.pallas import tpu as pltpu
from jax.experimental.pallas import tpu_sc as plsc

_H = 64
_LANES = 16
_TCHUNK = 512
_GSUB = 128


@functools.lru_cache(maxsize=None)
def _build(n_tokens, n_pos, n_age):
  info = plsc.get_sparse_core_info()
  nw = info.num_cores * info.num_subcores
  per_w = n_tokens // nw
  n_chunks = per_w // _TCHUNK
  nsub = _TCHUNK // _GSUB
  mesh = plsc.VectorSubcoreMesh(core_axis_name="c", subcore_axis_name="s")

  @functools.partial(
      pl.kernel,
      mesh=mesh,
      compiler_params=pltpu.CompilerParams(use_tc_tiling_on_sc=False),
      out_type=jax.ShapeDtypeStruct((n_tokens, _H), jnp.float32),
      scratch_types=[
          pltpu.VMEM((n_pos, _H), jnp.float32),
          pltpu.VMEM((n_age, _H), jnp.float32),
          pltpu.VMEM((2, _H), jnp.float32),
          pltpu.VMEM((2, _H), jnp.float32),
          pltpu.VMEM((_H,), jnp.float32),
          pltpu.VMEM((_H,), jnp.float32),
          pltpu.VMEM((nsub, _GSUB), jnp.int32),
          pltpu.VMEM((_TCHUNK,), jnp.int32),
          pltpu.VMEM((_TCHUNK,), jnp.int32),
          pltpu.VMEM((_TCHUNK,), jnp.int32),
          pltpu.VMEM((_TCHUNK,), jnp.int32),
          pltpu.VMEM((_TCHUNK, _H), jnp.float32),
          pltpu.SemaphoreType.DMA,
      ],
  )
  def emb_ln(wid_h, pid_h, aid_h, gid_h, sid_h,
             wtab_h, ptab_h, atab_h, gtab_h, stab_h, gam_h, bet_h,
             out_h,
             ptab, atab, gtab, stab, gam, bet,
             widx, pidx, aidx, gidx, sidx, rows, sem):
    w = lax.axis_index("s") * info.num_cores + lax.axis_index("c")
    base_w = w * per_w

    pltpu.sync_copy(ptab_h, ptab)
    pltpu.sync_copy(atab_h, atab)
    pltpu.sync_copy(gtab_h, gtab)
    pltpu.sync_copy(stab_h, stab)
    pltpu.sync_copy(gam_h, gam)
    pltpu.sync_copy(bet_h, bet)

    lane = lax.iota(jnp.int32, _LANES)
    perms = [lax.bitwise_xor(lane, jnp.int32(1 << p)) for p in range(4)]

    gdn = lax.GatherDimensionNumbers(
        offset_dims=(), collapsed_slice_dims=(0,), start_index_map=(0,))

    def allsum(v):
      for p in perms:
        v = v + lax.gather(v, p[:, None], gdn, (1,),
                           mode=lax.GatherScatterMode.PROMISE_IN_BOUNDS)
      return v

    gs0 = []
    gsd = []
    ssd = []
    gmk = []
    btk = []
    for k in range(4):
      sl = pl.ds(k * _LANES, _LANES)
      g0 = gtab[0, sl]
      g1 = gtab[1, sl]
      s0 = stab[0, sl]
      s1 = stab[1, sl]
      gs0.append(g0 + s0)
      gsd.append(g1 - g0)
      ssd.append(s1 - s0)
      gmk.append(gam[sl])
      btk.append(bet[sl])

    def chunk_body(c, carry):
      tb = base_w + c * _TCHUNK
      for j in range(nsub):
        pltpu.sync_copy(wid_h.at[pl.ds(tb + j * _GSUB, _GSUB)], widx.at[j])
      cps = [
          pltpu.async_copy(wtab_h.at[widx.at[j]],
                           rows.at[pl.ds(j * _GSUB, _GSUB)], sem)
          for j in range(nsub)
      ]
      pltpu.sync_copy(pid_h.at[pl.ds(tb, _TCHUNK)], pidx)
      pltpu.sync_copy(aid_h.at[pl.ds(tb, _TCHUNK)], aidx)
      pltpu.sync_copy(gid_h.at[pl.ds(tb, _TCHUNK)], gidx)
      pltpu.sync_copy(sid_h.at[pl.ds(tb, _TCHUNK)], sidx)
      for cp in cps:
        cp.wait()

      def token(t, pt, at, gf, sf):
        acc = []
        for k in range(4):
          sl = pl.ds(k * _LANES, _LANES)
          a = rows[t, sl] + ptab[pt, sl] + atab[at, sl]
          a = a + gs0[k] + gf * gsd[k] + sf * ssd[k]
          acc.append(a)
        s1 = (acc[0] + acc[1]) + (acc[2] + acc[3])
        s2 = (acc[0] * acc[0] + acc[1] * acc[1]) + (
            acc[2] * acc[2] + acc[3] * acc[3])
        tot = allsum(s1)
        tot2 = allsum(s2)
        mean = tot * (1.0 / _H)
        var = tot2 * (1.0 / _H) - mean * mean
        x = var + 1e-12
        xi = lax.bitcast_convert_type(x, jnp.int32)
        y = lax.bitcast_convert_type(
            jnp.int32(0x5F3759DF) - jnp.right_shift(xi, 1), jnp.float32)
        xh = x * 0.5
        y = y * (1.5 - xh * y * y)
        y = y * (1.5 - xh * y * y)
        y = y * (1.5 - xh * y * y)
        ms = mean * y
        for k in range(4):
          sl = pl.ds(k * _LANES, _LANES)
          rows[t, sl] = (acc[k] * y - ms) * gmk[k] + btk[k]

      def tbody(g, carry2):
        gb = g * _LANES
        pv = pidx[pl.ds(gb, _LANES)]
        av = aidx[pl.ds(gb, _LANES)]
        gv = gidx[pl.ds(gb, _LANES)].astype(jnp.float32)
        sv = sidx[pl.ds(gb, _LANES)].astype(jnp.float32)
        for u in range(_LANES):
          token(gb + u, pv[u], av[u],
                jnp.broadcast_to(gv[u], (_LANES,)),
                jnp.broadcast_to(sv[u], (_LANES,)))
        return carry2

      lax.fori_loop(0, _TCHUNK // _LANES, tbody, 0)
      pltpu.sync_copy(rows, out_h.at[pl.ds(tb, _TCHUNK)])
      return carry

    lax.fori_loop(0, n_chunks, chunk_body, 0)

  return emb_ln


def kernel(word_ids, seg_ids, posi_ids, age_ids, gender_ids,
           word_table, seg_table, age_table, gender_table, posi_table,
           gamma, beta):
  b, s = word_ids.shape
  n = b * s
  wi = word_ids.reshape(n).astype(jnp.int32)
  si = seg_ids.reshape(n).astype(jnp.int32)
  pi = posi_ids.reshape(n).astype(jnp.int32)
  ai = age_ids.reshape(n).astype(jnp.int32)
  gi = gender_ids.reshape(n).astype(jnp.int32)
  fn = _build(n, posi_table.shape[0], age_table.shape[0])
  out = fn(wi, pi, ai, gi, si,
           word_table, posi_table, age_table, gender_table, seg_table,
           gamma.astype(jnp.float32), beta.astype(jnp.float32))
  return out.reshape(b, s, _H)


# speedup vs baseline: 7.0067x; 1.0678x over previous
import functools

import jax
import jax.numpy as jnp
from jax import lax
from jax.experimental import pallas as pl
from jax.experimental.pallas import tpu as pltpu
from jax.experimental.pallas import tpu_sc as plsc

_H = 64
_LANES = 16
_TCHUNK = 256
_GSUB = 128
_NSUB = _TCHUNK // _GSUB


@functools.lru_cache(maxsize=None)
def _build(n_tokens, n_pos, n_age):
  info = plsc.get_sparse_core_info()
  nw = info.num_cores * info.num_subcores
  per_w = n_tokens // nw
  n_chunks = per_w // _TCHUNK
  mesh = plsc.VectorSubcoreMesh(core_axis_name="c", subcore_axis_name="s")

  idx_set = lambda: [
      pltpu.VMEM((_NSUB, _GSUB), jnp.int32),
      pltpu.VMEM((_TCHUNK,), jnp.int32),
      pltpu.VMEM((_TCHUNK,), jnp.int32),
      pltpu.VMEM((_TCHUNK,), jnp.int32),
      pltpu.VMEM((_TCHUNK,), jnp.int32),
  ]

  @functools.partial(
      pl.kernel,
      mesh=mesh,
      compiler_params=pltpu.CompilerParams(use_tc_tiling_on_sc=False),
      out_type=jax.ShapeDtypeStruct((n_tokens, _H), jnp.float32),
      scratch_types=[
          pltpu.VMEM((n_pos, _H), jnp.float32),
          pltpu.VMEM((n_age, _H), jnp.float32),
          pltpu.VMEM((2, _H), jnp.float32),
          pltpu.VMEM((2, _H), jnp.float32),
          pltpu.VMEM((_H,), jnp.float32),
          pltpu.VMEM((_H,), jnp.float32),
          [pltpu.VMEM((_TCHUNK, _H), jnp.float32) for _ in range(2)],
          [pltpu.VMEM((_TCHUNK, _H), jnp.float32) for _ in range(2)],
          [idx_set() for _ in range(2)],
          [pltpu.SemaphoreType.DMA for _ in range(2)],
          [pltpu.SemaphoreType.DMA for _ in range(2)],
          [pltpu.SemaphoreType.DMA for _ in range(2)],
      ],
  )
  def emb_ln(wid_h, pid_h, aid_h, gid_h, sid_h,
             wtab_h, ptab_h, atab_h, gtab_h, stab_h, gam_h, bet_h,
             out_h,
             ptab, atab, gtab, stab, gam, bet,
             rows, obuf, idxs, sem_g, sem_i, sem_o):
    w = lax.axis_index("s") * info.num_cores + lax.axis_index("c")
    base_w = w * per_w

    pltpu.sync_copy(ptab_h, ptab)
    pltpu.sync_copy(atab_h, atab)
    pltpu.sync_copy(gtab_h, gtab)
    pltpu.sync_copy(stab_h, stab)
    pltpu.sync_copy(gam_h, gam)
    pltpu.sync_copy(bet_h, bet)

    lane = lax.iota(jnp.int32, _LANES)
    perms = [lax.bitwise_xor(lane, jnp.int32(1 << p)) for p in range(4)]
    gdn = lax.GatherDimensionNumbers(
        offset_dims=(), collapsed_slice_dims=(0,), start_index_map=(0,))

    def allsum(v):
      for p in perms:
        v = v + lax.gather(v, p[:, None], gdn, (1,),
                           mode=lax.GatherScatterMode.PROMISE_IN_BOUNDS)
      return v

    gs0 = []
    gsd = []
    ssd = []
    gmk = []
    btk = []
    for k in range(4):
      sl = pl.ds(k * _LANES, _LANES)
      g0 = gtab[0, sl]
      g1 = gtab[1, sl]
      s0 = stab[0, sl]
      s1 = stab[1, sl]
      gs0.append(g0 + s0)
      gsd.append(g1 - g0)
      ssd.append(s1 - s0)
      gmk.append(gam[sl])
      btk.append(bet[sl])

    def idx_copies(c, s):
      tb = base_w + c * _TCHUNK
      widx, pidx, aidx, gidx, sidx = idxs[s]
      cps = [pltpu.make_async_copy(
          wid_h.at[pl.ds(tb + j * _GSUB, _GSUB)], widx.at[j], sem_i[s])
             for j in range(_NSUB)]
      for src, dst in ((pid_h, pidx), (aid_h, aidx),
                       (gid_h, gidx), (sid_h, sidx)):
        cps.append(pltpu.make_async_copy(
            src.at[pl.ds(tb, _TCHUNK)], dst, sem_i[s]))
      return cps

    def gather_copies(c, s):
      widx = idxs[s][0]
      return [pltpu.make_async_copy(
          wtab_h.at[widx.at[j]],
          rows[s].at[pl.ds(j * _GSUB, _GSUB)], sem_g[s])
              for j in range(_NSUB)]

    def out_copy(c, s):
      tb = base_w + c * _TCHUNK
      return pltpu.make_async_copy(
          obuf[s], out_h.at[pl.ds(tb, _TCHUNK)], sem_o[s])

    def token(rbuf, wbuf, t, pt, at, gf, sf):
      acc = []
      for k in range(4):
        sl = pl.ds(k * _LANES, _LANES)
        a = rbuf[t, sl] + ptab[pt, sl] + atab[at, sl]
        a = a + gs0[k] + gf * gsd[k] + sf * ssd[k]
        acc.append(a)
      s1 = (acc[0] + acc[1]) + (acc[2] + acc[3])
      s2 = (acc[0] * acc[0] + acc[1] * acc[1]) + (
          acc[2] * acc[2] + acc[3] * acc[3])
      tot = allsum(s1)
      tot2 = allsum(s2)
      mean = tot * (1.0 / _H)
      var = tot2 * (1.0 / _H) - mean * mean
      x = var + 1e-12
      xi = lax.bitcast_convert_type(x, jnp.int32)
      y = lax.bitcast_convert_type(
          jnp.int32(0x5F3759DF) - jnp.right_shift(xi, 1), jnp.float32)
      xh = x * 0.5
      y = y * (1.5 - xh * y * y)
      y = y * (1.5 - xh * y * y)
      y = y * (1.5 - xh * y * y)
      ms = mean * y
      for k in range(4):
        sl = pl.ds(k * _LANES, _LANES)
        wbuf[t, sl] = (acc[k] * y - ms) * gmk[k] + btk[k]

    def compute(s):
      _, pidx, aidx, gidx, sidx = idxs[s]

      def tbody(g, carry):
        gb = g * _LANES
        pv = pidx[pl.ds(gb, _LANES)]
        av = aidx[pl.ds(gb, _LANES)]
        gv = gidx[pl.ds(gb, _LANES)].astype(jnp.float32)
        sv = sidx[pl.ds(gb, _LANES)].astype(jnp.float32)
        for u in range(_LANES):
          token(rows[s], obuf[s], gb + u, pv[u], av[u],
                jnp.broadcast_to(gv[u], (_LANES,)),
                jnp.broadcast_to(sv[u], (_LANES,)))
        return carry

      lax.fori_loop(0, _TCHUNK // _LANES, tbody, 0)

    def do_chunk(c, s):
      ns = 1 - s

      @pl.when(c + 1 < n_chunks)
      def _():
        for cp in idx_copies(c + 1, ns):
          cp.wait()
        for cp in gather_copies(c + 1, ns):
          cp.start()

      @pl.when(c >= 2)
      def _():
        out_copy(c - 2, s).wait()

      for cp in gather_copies(c, s):
        cp.wait()
      compute(s)

      @pl.when(c + 2 < n_chunks)
      def _():
        for cp in idx_copies(c + 2, s):
          cp.start()

      out_copy(c, s).start()

    for cp in idx_copies(0, 0):
      cp.start()
      cp.wait()
    for cp in gather_copies(0, 0):
      cp.start()
    for cp in idx_copies(1, 1):
      cp.start()

    def pair_body(c2, carry):
      do_chunk(2 * c2, 0)
      do_chunk(2 * c2 + 1, 1)
      return carry

    lax.fori_loop(0, n_chunks // 2, pair_body, 0)
    out_copy(n_chunks - 2, 0).wait()
    out_copy(n_chunks - 1, 1).wait()

  return emb_ln


def kernel(word_ids, seg_ids, posi_ids, age_ids, gender_ids,
           word_table, seg_table, age_table, gender_table, posi_table,
           gamma, beta):
  b, s = word_ids.shape
  n = b * s
  wi = word_ids.reshape(n).astype(jnp.int32)
  si = seg_ids.reshape(n).astype(jnp.int32)
  pi = posi_ids.reshape(n).astype(jnp.int32)
  ai = age_ids.reshape(n).astype(jnp.int32)
  gi = gender_ids.reshape(n).astype(jnp.int32)
  fn = _build(n, posi_table.shape[0], age_table.shape[0])
  out = fn(wi, pi, ai, gi, si,
           word_table, posi_table, age_table, gender_table, seg_table,
           gamma.astype(jnp.float32), beta.astype(jnp.float32))
  return out.reshape(b, s, _H)
